# Initial kernel scaffold; baseline (speedup 1.0000x reference)
import jax
import jax.numpy as jnp
from jax.experimental import pallas as pl


def kernel(seq, table):
    raise NotImplementedError("write your pallas kernel here")


# speedup vs baseline: 73.6393x; 73.6393x over previous
import jax
import jax.numpy as jnp
from jax import lax
from jax.experimental import pallas as pl
from jax.experimental.pallas import tpu as pltpu
from jax.experimental.pallas import tpu_sc as plsc

_NC = 2
_NS = 16
_NW = _NC * _NS
_LANES = 16
_CHUNK = 16384


def _body(seq_hbm, tbl_hbm, out_hbm, tbl_v, idx_v, r0, r1, r2, r3):
    wid = lax.axis_index("s") * _NC + lax.axis_index("c")
    per_w = seq_hbm.shape[0] // _NW
    base = wid * per_w
    pltpu.sync_copy(tbl_hbm, tbl_v)
    rows = [r0, r1, r2, r3]
    for c in range(per_w // _CHUNK):
        cb = base + c * _CHUNK

        pltpu.sync_copy(seq_hbm.at[pl.ds(cb, _CHUNK)], idx_v)

        def inner(k, _):
            off = pl.multiple_of(k * _LANES, _LANES)
            idx4 = idx_v[pl.ds(off, _LANES)] << 2
            for j in range(4):
                rows[j][pl.ds(off, _LANES)] = plsc.load_gather(
                    tbl_v, [idx4 | j])
            return 0

        lax.fori_loop(0, _CHUNK // _LANES, inner, 0)

        for j in range(4):
            pltpu.sync_copy(rows[j], out_hbm.at[j, pl.ds(cb, _CHUNK)])


def kernel(seq, table):
    L = seq.shape[0]
    seq = seq.astype(jnp.int32)
    tbl = jnp.zeros((32,), jnp.float32).at[:20].set(
        table.astype(jnp.float32).reshape(-1))
    mesh = plsc.VectorSubcoreMesh(core_axis_name="c", subcore_axis_name="s")
    f = pl.kernel(
        _body,
        out_type=jax.ShapeDtypeStruct((4, L), jnp.float32),
        mesh=mesh,
        compiler_params=pltpu.CompilerParams(needs_layout_passes=False),
        scratch_types=[
            pltpu.VMEM((32,), jnp.float32),
            pltpu.VMEM((_CHUNK,), jnp.int32),
        ] + [pltpu.VMEM((_CHUNK,), jnp.float32) for _ in range(4)],
    )
    return f(seq, tbl)
